# baseline (speedup 1.0000x reference)
import functools

import jax
import jax.numpy as jnp
from jax import lax
from jax.experimental import pallas as pl
from jax.experimental.pallas import tpu as pltpu
from jax.experimental.pallas import tpu_sc as plsc

N = 10000
E = 160000
IN_C = 256
OUT_C = 256
H = 4
HID = 1024
FB = 128
NFB = HID // FB
ALPHA = 0.2

R1 = 1000
R4 = 1000

NC, NS, L = 2, 16, 16
NW = NC * NS
E_PAD = 163840
CH = 128
NCH = E_PAD // CH
CH2 = NCH // NW
EPT2 = E_PAD // NW
CG = 64
NCG = E_PAD // CG
CH3 = NCG // NS
EPT3 = E_PAD // NS
NPT = N // NS


def _k1_body(h_ref, w_ref, a_ref, wh_ref, s_ref, m_ref, m_acc):
    i = pl.program_id(0)
    x = h_ref[...]
    y = lax.dot_general(x, w_ref[...], (((1,), (1,)), ((), ())),
                        preferred_element_type=jnp.float32)
    for fb in range(NFB):
        wh_ref[fb] = y[:, fb * FB:(fb + 1) * FB]
    s = lax.dot_general(y, a_ref[...], (((1,), (0,)), ((), ())),
                        preferred_element_type=jnp.float32)
    s_ref[...] = s
    bm = jnp.max(s, axis=0, keepdims=True)
    prev = m_acc[...]
    cur = jnp.where(i == 0, bm, jnp.maximum(prev, bm))
    m_acc[...] = cur
    m12 = cur[:, :H] + cur[:, H:]
    mlr = jnp.maximum(m12, ALPHA * m12)
    m_ref[...] = jnp.concatenate([mlr, mlr], axis=1)


def _k1(h, W, A):
    return pl.pallas_call(
        _k1_body,
        grid=(N // R1,),
        in_specs=[
            pl.BlockSpec((R1, IN_C), lambda i: (i, 0)),
            pl.BlockSpec((HID, IN_C), lambda i: (0, 0)),
            pl.BlockSpec((HID, 2 * H), lambda i: (0, 0)),
        ],
        out_specs=[
            pl.BlockSpec((NFB, R1, FB), lambda i: (0, i, 0)),
            pl.BlockSpec((R1, 2 * H), lambda i: (i, 0)),
            pl.BlockSpec((1, 2 * H), lambda i: (0, 0)),
        ],
        out_shape=[
            jax.ShapeDtypeStruct((NFB, N, FB), jnp.float32),
            jax.ShapeDtypeStruct((N, 2 * H), jnp.float32),
            jax.ShapeDtypeStruct((1, 2 * H), jnp.float32),
        ],
        scratch_shapes=[pltpu.VMEM((1, 2 * H), jnp.float32)],
    )(h, W, A)


def _k2_body(s_hbm, row2_hbm, col2_hbm, m_hbm, z8_hbm,
             ew_hbm, esum_hbm,
             s_v, row_v, col_v, m_v, ew_v, esrc_v, efl_v, esum_sh):
    c = lax.axis_index("c")
    s = lax.axis_index("s")
    w = c * NS + s

    pltpu.sync_copy(s_hbm, s_v)
    pltpu.sync_copy(row2_hbm.at[pl.ds(w * CH2, CH2)], row_v)
    pltpu.sync_copy(col2_hbm.at[pl.ds(w * CH2, CH2)], col_v)
    pltpu.sync_copy(m_hbm, m_v)
    pltpu.sync_copy(z8_hbm.at[pl.ds(0, CH)], esrc_v)
    @pl.when(s < 10)
    def _():
        pltpu.sync_copy(z8_hbm.at[pl.ds(s * 1000, 1000)],
                        esum_sh.at[pl.ds(s * 1000, 1000)])
    plsc.subcore_barrier()

    iota16 = lax.iota(jnp.int32, L)

    def chunk_body(ch, _):
        def group_body(g, _):
            ir = row_v[ch, pl.ds(g * L, L)]
            ic = col_v[ch, pl.ds(g * L, L)]
            gid = (w * EPT2 + ch * CH + g * L) + iota16
            valid = gid < E
            for h in range(H):
                v1 = plsc.load_gather(s_v, [ir, jnp.full((L,), h, jnp.int32)])
                v2 = plsc.load_gather(s_v, [ic, jnp.full((L,), H + h, jnp.int32)])
                p = v1 + v2
                e = jnp.exp(jnp.maximum(p, ALPHA * p) - m_v[h])
                e = jnp.where(valid, e, 0.0)
                ew_v[h, pl.ds(ch * CH + g * L, L)] = e
                plsc.store_scatter(
                    esrc_v, [g * L + iota16, jnp.full((L,), h, jnp.int32)], e)
            return 0
        lax.fori_loop(0, CH // L, group_body, 0)
        pltpu.sync_copy(esrc_v, esum_sh.at[row_v.at[ch]], add=True)
        return 0

    lax.fori_loop(0, CH2, chunk_body, 0)

    for h in range(H):
        pltpu.sync_copy(ew_v.at[h], ew_hbm.at[h, pl.ds(w * EPT2, EPT2)])
    plsc.subcore_barrier()
    @pl.when(s < 10)
    def _():
        pltpu.sync_copy(esum_sh.at[pl.ds(s * 1000, 1000)], efl_v)
        pltpu.sync_copy(efl_v, esum_hbm.at[c, pl.ds(s * 1000, 1000)])


def _k2(S, row2, col2, m_bcast, zeros8):
    mesh = plsc.VectorSubcoreMesh(core_axis_name="c", subcore_axis_name="s")
    f = functools.partial(
        pl.kernel,
        out_type=[
            jax.ShapeDtypeStruct((H, E_PAD), jnp.float32),
            jax.ShapeDtypeStruct((NC, N, 2 * H), jnp.float32),
        ],
        mesh=mesh,
        compiler_params=pltpu.CompilerParams(needs_layout_passes=False, use_tc_tiling_on_sc=False),
        scratch_types=[
            pltpu.VMEM((N, 2 * H), jnp.float32),
            pltpu.VMEM((CH2, CH), jnp.int32),
            pltpu.VMEM((CH2, CH), jnp.int32),
            pltpu.VMEM((H, L), jnp.float32),
            pltpu.VMEM((H, EPT2), jnp.float32),
            pltpu.VMEM((CH, 2 * H), jnp.float32),
            pltpu.VMEM((1000, 2 * H), jnp.float32),
            pltpu.VMEM_SHARED((N, 2 * H), jnp.float32),
        ],
    )(_k2_body)
    return f(S, row2, col2, m_bcast, zeros8)


def _k3_body(wh_hbm, row2_hbm, col2_hbm, ew_hbm, z128_hbm,
             out_hbm,
             row_v, col_v, e_v, rb0, rb1, sg0, sg1, ss0, ss1, acc_sh):
    c = lax.axis_index("c")
    s = lax.axis_index("s")

    pltpu.sync_copy(row2_hbm.at[pl.ds(s * CH3, CH3)], row_v)
    pltpu.sync_copy(col2_hbm.at[pl.ds(s * CH3, CH3)], col_v)

    def scale(rb, ch):
        def edge4(k, _):
            for u in range(4):
                i = k * 4 + u
                e16 = plsc.load_gather(
                    e_v, [jnp.full((L,), ch * CG + i, jnp.int32)])
                for r in range(FB // L):
                    sl = pl.ds(r * L, L)
                    rb[i, sl] = rb[i, sl] * e16
            return 0
        lax.fori_loop(0, CG // 4, edge4, 0)

    for fbi in range(NFB // NC):
        fb = c * (NFB // NC) + fbi
        head = fb // (NFB // H)
        @pl.when(s < 10)
        def _():
            pltpu.sync_copy(z128_hbm.at[pl.ds(s * 1000, 1000)],
                            acc_sh.at[pl.ds(s * 1000, 1000)])
        plsc.subcore_barrier()
        pltpu.sync_copy(ew_hbm.at[head, pl.ds(s * EPT3, EPT3)], e_v)

        pltpu.async_copy(wh_hbm.at[fb].at[col_v.at[0]], rb0, sg0)

        def pair_body(j, _):
            ch0 = j * 2
            ch1 = ch0 + 1
            @pl.when(j > 0)
            def _():
                pltpu.make_async_copy(
                    rb1, acc_sh.at[row_v.at[ch1 - 2]], ss1).wait()
            pltpu.async_copy(wh_hbm.at[fb].at[col_v.at[ch1]], rb1, sg1)
            pltpu.make_async_copy(wh_hbm.at[fb].at[col_v.at[ch0]], rb0, sg0).wait()
            scale(rb0, ch0)
            pltpu.async_copy(rb0, acc_sh.at[row_v.at[ch0]], ss0, add=True)
            pltpu.make_async_copy(wh_hbm.at[fb].at[col_v.at[ch1]], rb1, sg1).wait()
            scale(rb1, ch1)
            pltpu.async_copy(rb1, acc_sh.at[row_v.at[ch1]], ss1, add=True)
            pltpu.make_async_copy(rb0, acc_sh.at[row_v.at[ch0]], ss0).wait()
            @pl.when(ch0 + 2 < CH3)
            def _():
                pltpu.async_copy(wh_hbm.at[fb].at[col_v.at[ch0 + 2]], rb0, sg0)
            return 0

        lax.fori_loop(0, CH3 // 2, pair_body, 0)
        pltpu.make_async_copy(rb1, acc_sh.at[row_v.at[CH3 - 1]], ss1).wait()
        plsc.subcore_barrier()
        @pl.when(s < 10)
        def _():
            for j, sz in enumerate([CG] * 15 + [1000 - 15 * CG]):
                r0 = s * 1000 + j * CG
                pltpu.sync_copy(acc_sh.at[pl.ds(r0, sz)], rb0.at[pl.ds(0, sz)])
                pltpu.sync_copy(rb0.at[pl.ds(0, sz)],
                                out_hbm.at[fb, pl.ds(r0, sz)])
        plsc.subcore_barrier()


def _k3(wh, row3, col3, ew, zeros128):
    mesh = plsc.VectorSubcoreMesh(core_axis_name="c", subcore_axis_name="s")
    f = functools.partial(
        pl.kernel,
        out_type=jax.ShapeDtypeStruct((NFB, N, FB), jnp.float32),
        mesh=mesh,
        compiler_params=pltpu.CompilerParams(needs_layout_passes=False, use_tc_tiling_on_sc=False),
        scratch_types=[
            pltpu.VMEM((CH3, CG), jnp.int32),
            pltpu.VMEM((CH3, CG), jnp.int32),
            pltpu.VMEM((EPT3,), jnp.float32),
            pltpu.VMEM((CG, FB), jnp.float32),
            pltpu.VMEM((CG, FB), jnp.float32),
            pltpu.SemaphoreType.DMA,
            pltpu.SemaphoreType.DMA,
            pltpu.SemaphoreType.DMA,
            pltpu.SemaphoreType.DMA,
            pltpu.VMEM_SHARED((N, FB), jnp.float32),
        ],
    )(_k3_body)
    return f(wh, row3, col3, ew, zeros128)


def _k4_body(es_ref, g_ref, b_ref, fw_ref, fbias_ref, hacc_ref, wh_ref, out_ref):
    es = es_ref[0] + es_ref[1]
    cols = []
    for fb in range(NFB):
        head = fb // (NFB // H)
        x = hacc_ref[fb] / (es[:, head:head + 1] + 1e-15) + wh_ref[fb]
        cols.append(x)
    X = jnp.concatenate(cols, axis=1)
    mean = jnp.mean(X, axis=1, keepdims=True)
    xc = X - mean
    var = jnp.mean(xc * xc, axis=1, keepdims=True)
    Xn = xc * lax.rsqrt(var + 1e-5) * g_ref[...] + b_ref[...]
    Y = lax.dot_general(Xn, fw_ref[...], (((1,), (1,)), ((), ())),
                        preferred_element_type=jnp.float32) + fbias_ref[...]
    out_ref[...] = jnp.maximum(Y, 0.0)


def _k4(esum2, gamma, beta, ffn_W, ffn_b, hacc, wh):
    return pl.pallas_call(
        _k4_body,
        grid=(N // R4,),
        in_specs=[
            pl.BlockSpec((2, R4, 2 * H), lambda i: (0, i, 0)),
            pl.BlockSpec((1, HID), lambda i: (0, 0)),
            pl.BlockSpec((1, HID), lambda i: (0, 0)),
            pl.BlockSpec((HID, HID), lambda i: (0, 0)),
            pl.BlockSpec((1, HID), lambda i: (0, 0)),
            pl.BlockSpec((NFB, R4, FB), lambda i: (0, i, 0)),
            pl.BlockSpec((NFB, R4, FB), lambda i: (0, i, 0)),
        ],
        out_specs=pl.BlockSpec((R4, HID), lambda i: (i, 0)),
        out_shape=jax.ShapeDtypeStruct((N, HID), jnp.float32),
    )(esum2, gamma.reshape(1, HID), beta.reshape(1, HID), ffn_W,
      ffn_b.reshape(1, HID), hacc, wh)


def kernel(h, edge_index, W, attn_fc, ln_gamma, ln_beta, ffn_W, ffn_b):
    row = edge_index[0].astype(jnp.int32)
    col = edge_index[1].astype(jnp.int32)
    row2 = jnp.pad(row, (0, E_PAD - E)).reshape(NCH, CH)
    col2 = jnp.pad(col, (0, E_PAD - E)).reshape(NCH, CH)
    a1 = attn_fc[:, :OUT_C, 0]
    a2 = attn_fc[:, OUT_C:, 0]
    A = jnp.zeros((HID, 2 * H), jnp.float32)
    hh = jnp.arange(HID) // OUT_C
    A = A.at[jnp.arange(HID), hh].set(a1.reshape(-1))
    A = A.at[jnp.arange(HID), H + hh].set(a2.reshape(-1))

    wh, S, m8 = _k1(h, W, A)
    m_bcast = jnp.broadcast_to(m8[0, :H, None], (H, L))
    zeros8 = jnp.zeros((N, 2 * H), jnp.float32)
    zeros128 = jnp.zeros((N, FB), jnp.float32)

    ew, esum2 = _k2(S, row2, col2, m_bcast, zeros8)
    row3 = row2.reshape(NCG, CG)
    col3 = col2.reshape(NCG, CG)
    hacc = _k3(wh, row3, col3, ew, zeros128)
    return _k4(esum2, ln_gamma, ln_beta, ffn_W, ffn_b, hacc, wh)


# speedup vs baseline: 1.1349x; 1.1349x over previous
import functools

import jax
import jax.numpy as jnp
from jax import lax
from jax.experimental import pallas as pl
from jax.experimental.pallas import tpu as pltpu
from jax.experimental.pallas import tpu_sc as plsc

N = 10000
E = 160000
IN_C = 256
OUT_C = 256
H = 4
HID = 1024
FB = 128
NFB = HID // FB
ALPHA = 0.2

R1 = 1000
R4 = 1000

NC, NS, L = 2, 16, 16
NW = NC * NS
E_PAD = 163840
CH = 128
NCH = E_PAD // CH
CH2 = NCH // NW
EPT2 = E_PAD // NW
CG = 64
NCG = E_PAD // CG
CH3 = NCG // NS
EPT3 = E_PAD // NS
NPT = N // NS


def _k1_body(h_ref, w_ref, a_ref, wh_ref, s_ref, m_ref, m_acc):
    i = pl.program_id(0)
    x = h_ref[...]
    y = lax.dot_general(x, w_ref[...], (((1,), (1,)), ((), ())),
                        preferred_element_type=jnp.float32)
    for fb in range(NFB):
        wh_ref[fb] = y[:, fb * FB:(fb + 1) * FB]
    s = lax.dot_general(y, a_ref[...], (((1,), (0,)), ((), ())),
                        preferred_element_type=jnp.float32)
    s_ref[...] = s
    bm = jnp.max(s, axis=0, keepdims=True)
    prev = m_acc[...]
    cur = jnp.where(i == 0, bm, jnp.maximum(prev, bm))
    m_acc[...] = cur
    m12 = cur[:, :H] + cur[:, H:]
    mlr = jnp.maximum(m12, ALPHA * m12)
    m_ref[...] = jnp.concatenate([mlr, mlr], axis=1)


def _k1(h, W, A):
    return pl.pallas_call(
        _k1_body,
        grid=(N // R1,),
        in_specs=[
            pl.BlockSpec((R1, IN_C), lambda i: (i, 0)),
            pl.BlockSpec((HID, IN_C), lambda i: (0, 0)),
            pl.BlockSpec((HID, 2 * H), lambda i: (0, 0)),
        ],
        out_specs=[
            pl.BlockSpec((NFB, R1, FB), lambda i: (0, i, 0)),
            pl.BlockSpec((R1, 2 * H), lambda i: (i, 0)),
            pl.BlockSpec((1, 2 * H), lambda i: (0, 0)),
        ],
        out_shape=[
            jax.ShapeDtypeStruct((NFB, N, FB), jnp.float32),
            jax.ShapeDtypeStruct((N, 2 * H), jnp.float32),
            jax.ShapeDtypeStruct((1, 2 * H), jnp.float32),
        ],
        scratch_shapes=[pltpu.VMEM((1, 2 * H), jnp.float32)],
    )(h, W, A)


def _k2_body(s_hbm, row2_hbm, col2_hbm, m_hbm, z8_hbm,
             ew_hbm, esum_hbm,
             s_v, row_v, col_v, m_v, ew_v, esrc_v, efl_v, esum_sh):
    c = lax.axis_index("c")
    s = lax.axis_index("s")
    w = c * NS + s

    pltpu.sync_copy(s_hbm, s_v)
    pltpu.sync_copy(row2_hbm.at[pl.ds(w * CH2, CH2)], row_v)
    pltpu.sync_copy(col2_hbm.at[pl.ds(w * CH2, CH2)], col_v)
    pltpu.sync_copy(m_hbm, m_v)
    pltpu.sync_copy(z8_hbm.at[pl.ds(0, CH)], esrc_v)
    @pl.when(s < 10)
    def _():
        pltpu.sync_copy(z8_hbm.at[pl.ds(s * 1000, 1000)],
                        esum_sh.at[pl.ds(s * 1000, 1000)])
    plsc.subcore_barrier()

    iota16 = lax.iota(jnp.int32, L)

    def chunk_body(ch, _):
        def group_body(g, _):
            ir = row_v[ch, pl.ds(g * L, L)]
            ic = col_v[ch, pl.ds(g * L, L)]
            gid = (w * EPT2 + ch * CH + g * L) + iota16
            valid = gid < E
            for h in range(H):
                v1 = plsc.load_gather(s_v, [ir, jnp.full((L,), h, jnp.int32)])
                v2 = plsc.load_gather(s_v, [ic, jnp.full((L,), H + h, jnp.int32)])
                p = v1 + v2
                e = jnp.exp(jnp.maximum(p, ALPHA * p) - m_v[h])
                e = jnp.where(valid, e, 0.0)
                ew_v[h, pl.ds(ch * CH + g * L, L)] = e
                plsc.store_scatter(
                    esrc_v, [g * L + iota16, jnp.full((L,), h, jnp.int32)], e)
            return 0
        lax.fori_loop(0, CH // L, group_body, 0)
        pltpu.sync_copy(esrc_v, esum_sh.at[row_v.at[ch]], add=True)
        return 0

    lax.fori_loop(0, CH2, chunk_body, 0)

    for h in range(H):
        pltpu.sync_copy(ew_v.at[h], ew_hbm.at[h, pl.ds(w * EPT2, EPT2)])
    plsc.subcore_barrier()
    @pl.when(s < 10)
    def _():
        pltpu.sync_copy(esum_sh.at[pl.ds(s * 1000, 1000)], efl_v)
        pltpu.sync_copy(efl_v, esum_hbm.at[c, pl.ds(s * 1000, 1000)])


def _k2(S, row2, col2, m_bcast, zeros8):
    mesh = plsc.VectorSubcoreMesh(core_axis_name="c", subcore_axis_name="s")
    f = functools.partial(
        pl.kernel,
        out_type=[
            jax.ShapeDtypeStruct((H, E_PAD), jnp.float32),
            jax.ShapeDtypeStruct((NC, N, 2 * H), jnp.float32),
        ],
        mesh=mesh,
        compiler_params=pltpu.CompilerParams(needs_layout_passes=False, use_tc_tiling_on_sc=False),
        scratch_types=[
            pltpu.VMEM((N, 2 * H), jnp.float32),
            pltpu.VMEM((CH2, CH), jnp.int32),
            pltpu.VMEM((CH2, CH), jnp.int32),
            pltpu.VMEM((H, L), jnp.float32),
            pltpu.VMEM((H, EPT2), jnp.float32),
            pltpu.VMEM((CH, 2 * H), jnp.float32),
            pltpu.VMEM((1000, 2 * H), jnp.float32),
            pltpu.VMEM_SHARED((N, 2 * H), jnp.float32),
        ],
    )(_k2_body)
    return f(S, row2, col2, m_bcast, zeros8)


def _k3_body(wh_hbm, row2_hbm, col2_hbm, ew_hbm, z128_hbm,
             out_hbm,
             row_v, col_v, e_v, rb0, rb1, sg0, sg1, ss0, ss1, acc_sh):
    c = lax.axis_index("c")
    s = lax.axis_index("s")

    pltpu.sync_copy(row2_hbm.at[pl.ds(s * CH3, CH3)], row_v)
    pltpu.sync_copy(col2_hbm.at[pl.ds(s * CH3, CH3)], col_v)

    def scale(rb, ch):
        @plsc.parallel_loop(0, CG, unroll=4)
        def _(i):
            e16 = plsc.load_gather(
                e_v, [jnp.full((L,), ch * CG + i, jnp.int32)])
            for r in range(FB // L):
                sl = pl.ds(r * L, L)
                rb[i, sl] = rb[i, sl] * e16

    for fbi in range(NFB // NC):
        fb = c * (NFB // NC) + fbi
        head = fb // (NFB // H)
        @pl.when(s < 10)
        def _():
            pltpu.sync_copy(z128_hbm.at[pl.ds(s * 1000, 1000)],
                            acc_sh.at[pl.ds(s * 1000, 1000)])
        plsc.subcore_barrier()
        pltpu.sync_copy(ew_hbm.at[head, pl.ds(s * EPT3, EPT3)], e_v)

        pltpu.async_copy(wh_hbm.at[fb].at[col_v.at[0]], rb0, sg0)

        def pair_body(j, _):
            ch0 = j * 2
            ch1 = ch0 + 1
            pltpu.async_copy(wh_hbm.at[fb].at[col_v.at[ch1]], rb1, sg1)
            pltpu.make_async_copy(wh_hbm.at[fb].at[col_v.at[ch0]], rb0, sg0).wait()
            scale(rb0, ch0)
            pltpu.sync_copy(rb0, acc_sh.at[row_v.at[ch0]], add=True)
            @pl.when(ch0 + 2 < CH3)
            def _():
                pltpu.async_copy(wh_hbm.at[fb].at[col_v.at[ch0 + 2]], rb0, sg0)
            pltpu.make_async_copy(wh_hbm.at[fb].at[col_v.at[ch1]], rb1, sg1).wait()
            scale(rb1, ch1)
            pltpu.sync_copy(rb1, acc_sh.at[row_v.at[ch1]], add=True)
            return 0

        lax.fori_loop(0, CH3 // 2, pair_body, 0)
        plsc.subcore_barrier()
        @pl.when(s < 10)
        def _():
            for j, sz in enumerate([CG] * 15 + [1000 - 15 * CG]):
                r0 = s * 1000 + j * CG
                pltpu.sync_copy(acc_sh.at[pl.ds(r0, sz)], rb0.at[pl.ds(0, sz)])
                pltpu.sync_copy(rb0.at[pl.ds(0, sz)],
                                out_hbm.at[fb, pl.ds(r0, sz)])
        plsc.subcore_barrier()


def _k3(wh, row3, col3, ew, zeros128):
    mesh = plsc.VectorSubcoreMesh(core_axis_name="c", subcore_axis_name="s")
    f = functools.partial(
        pl.kernel,
        out_type=jax.ShapeDtypeStruct((NFB, N, FB), jnp.float32),
        mesh=mesh,
        compiler_params=pltpu.CompilerParams(needs_layout_passes=False, use_tc_tiling_on_sc=False),
        scratch_types=[
            pltpu.VMEM((CH3, CG), jnp.int32),
            pltpu.VMEM((CH3, CG), jnp.int32),
            pltpu.VMEM((EPT3,), jnp.float32),
            pltpu.VMEM((CG, FB), jnp.float32),
            pltpu.VMEM((CG, FB), jnp.float32),
            pltpu.SemaphoreType.DMA,
            pltpu.SemaphoreType.DMA,
            pltpu.SemaphoreType.DMA,
            pltpu.SemaphoreType.DMA,
            pltpu.VMEM_SHARED((N, FB), jnp.float32),
        ],
    )(_k3_body)
    return f(wh, row3, col3, ew, zeros128)


def _k4_body(es_ref, g_ref, b_ref, fw_ref, fbias_ref, hacc_ref, wh_ref, out_ref):
    es = es_ref[0] + es_ref[1]
    cols = []
    for fb in range(NFB):
        head = fb // (NFB // H)
        x = hacc_ref[fb] / (es[:, head:head + 1] + 1e-15) + wh_ref[fb]
        cols.append(x)
    X = jnp.concatenate(cols, axis=1)
    mean = jnp.mean(X, axis=1, keepdims=True)
    xc = X - mean
    var = jnp.mean(xc * xc, axis=1, keepdims=True)
    Xn = xc * lax.rsqrt(var + 1e-5) * g_ref[...] + b_ref[...]
    Y = lax.dot_general(Xn, fw_ref[...], (((1,), (1,)), ((), ())),
                        preferred_element_type=jnp.float32) + fbias_ref[...]
    out_ref[...] = jnp.maximum(Y, 0.0)


def _k4(esum2, gamma, beta, ffn_W, ffn_b, hacc, wh):
    return pl.pallas_call(
        _k4_body,
        grid=(N // R4,),
        in_specs=[
            pl.BlockSpec((2, R4, 2 * H), lambda i: (0, i, 0)),
            pl.BlockSpec((1, HID), lambda i: (0, 0)),
            pl.BlockSpec((1, HID), lambda i: (0, 0)),
            pl.BlockSpec((HID, HID), lambda i: (0, 0)),
            pl.BlockSpec((1, HID), lambda i: (0, 0)),
            pl.BlockSpec((NFB, R4, FB), lambda i: (0, i, 0)),
            pl.BlockSpec((NFB, R4, FB), lambda i: (0, i, 0)),
        ],
        out_specs=pl.BlockSpec((R4, HID), lambda i: (i, 0)),
        out_shape=jax.ShapeDtypeStruct((N, HID), jnp.float32),
    )(esum2, gamma.reshape(1, HID), beta.reshape(1, HID), ffn_W,
      ffn_b.reshape(1, HID), hacc, wh)


def kernel(h, edge_index, W, attn_fc, ln_gamma, ln_beta, ffn_W, ffn_b):
    row = edge_index[0].astype(jnp.int32)
    col = edge_index[1].astype(jnp.int32)
    row2 = jnp.pad(row, (0, E_PAD - E)).reshape(NCH, CH)
    col2 = jnp.pad(col, (0, E_PAD - E)).reshape(NCH, CH)
    a1 = attn_fc[:, :OUT_C, 0]
    a2 = attn_fc[:, OUT_C:, 0]
    A = jnp.zeros((HID, 2 * H), jnp.float32)
    hh = jnp.arange(HID) // OUT_C
    A = A.at[jnp.arange(HID), hh].set(a1.reshape(-1))
    A = A.at[jnp.arange(HID), H + hh].set(a2.reshape(-1))

    wh, S, m8 = _k1(h, W, A)
    m_bcast = jnp.broadcast_to(m8[0, :H, None], (H, L))
    zeros8 = jnp.zeros((N, 2 * H), jnp.float32)
    zeros128 = jnp.zeros((N, FB), jnp.float32)

    ew, esum2 = _k2(S, row2, col2, m_bcast, zeros8)
    row3 = row2.reshape(NCG, CG)
    col3 = col2.reshape(NCG, CG)
    hacc = _k3(wh, row3, col3, ew, zeros128)
    return _k4(esum2, ln_gamma, ln_beta, ffn_W, ffn_b, hacc, wh)


# speedup vs baseline: 1.1358x; 1.0008x over previous
import functools

import jax
import jax.numpy as jnp
from jax import lax
from jax.experimental import pallas as pl
from jax.experimental.pallas import tpu as pltpu
from jax.experimental.pallas import tpu_sc as plsc

N = 10000
E = 160000
IN_C = 256
OUT_C = 256
H = 4
HID = 1024
FB = 128
NFB = HID // FB
ALPHA = 0.2

R1 = 1000
R4 = 1000

NC, NS, L = 2, 16, 16
NW = NC * NS
E_PAD = 163840
CH = 128
NCH = E_PAD // CH
CH2 = NCH // NW
EPT2 = E_PAD // NW
CG = 64
NCG = E_PAD // CG
CH3 = NCG // NS
EPT3 = E_PAD // NS
NPT = N // NS


def _k1_body(h_ref, w_ref, a_ref, wh_ref, s_ref, m_ref, m_acc):
    i = pl.program_id(0)
    x = h_ref[...]
    y = lax.dot_general(x, w_ref[...], (((1,), (1,)), ((), ())),
                        preferred_element_type=jnp.float32)
    for fb in range(NFB):
        wh_ref[fb] = y[:, fb * FB:(fb + 1) * FB]
    s = lax.dot_general(y, a_ref[...], (((1,), (0,)), ((), ())),
                        preferred_element_type=jnp.float32)
    s_ref[...] = s
    bm = jnp.max(s, axis=0, keepdims=True)
    prev = m_acc[...]
    cur = jnp.where(i == 0, bm, jnp.maximum(prev, bm))
    m_acc[...] = cur
    m12 = cur[:, :H] + cur[:, H:]
    mlr = jnp.maximum(m12, ALPHA * m12)
    m_ref[...] = jnp.concatenate([mlr, mlr], axis=1)


def _k1(h, W, A):
    return pl.pallas_call(
        _k1_body,
        grid=(N // R1,),
        in_specs=[
            pl.BlockSpec((R1, IN_C), lambda i: (i, 0)),
            pl.BlockSpec((HID, IN_C), lambda i: (0, 0)),
            pl.BlockSpec((HID, 2 * H), lambda i: (0, 0)),
        ],
        out_specs=[
            pl.BlockSpec((NFB, R1, FB), lambda i: (0, i, 0)),
            pl.BlockSpec((R1, 2 * H), lambda i: (i, 0)),
            pl.BlockSpec((1, 2 * H), lambda i: (0, 0)),
        ],
        out_shape=[
            jax.ShapeDtypeStruct((NFB, N, FB), jnp.float32),
            jax.ShapeDtypeStruct((N, 2 * H), jnp.float32),
            jax.ShapeDtypeStruct((1, 2 * H), jnp.float32),
        ],
        scratch_shapes=[pltpu.VMEM((1, 2 * H), jnp.float32)],
    )(h, W, A)


def _k2_body(s_hbm, row2_hbm, col2_hbm, m_hbm, z8_hbm,
             ew_hbm, esum_hbm,
             s_v, row_v, col_v, m_v, ew_v, esrc_v, efl_v, esum_sh):
    c = lax.axis_index("c")
    s = lax.axis_index("s")
    w = c * NS + s

    pltpu.sync_copy(s_hbm, s_v)
    pltpu.sync_copy(row2_hbm.at[pl.ds(w * CH2, CH2)], row_v)
    pltpu.sync_copy(col2_hbm.at[pl.ds(w * CH2, CH2)], col_v)
    pltpu.sync_copy(m_hbm, m_v)
    pltpu.sync_copy(z8_hbm.at[pl.ds(0, CH)], esrc_v)
    @pl.when(s < 10)
    def _():
        pltpu.sync_copy(z8_hbm.at[pl.ds(s * 1000, 1000)],
                        esum_sh.at[pl.ds(s * 1000, 1000)])
    plsc.subcore_barrier()

    iota16 = lax.iota(jnp.int32, L)

    def chunk_body(ch, _):
        def group_body(g, _):
            ir = row_v[ch, pl.ds(g * L, L)]
            ic = col_v[ch, pl.ds(g * L, L)]
            gid = (w * EPT2 + ch * CH + g * L) + iota16
            valid = gid < E
            for h in range(H):
                v1 = plsc.load_gather(s_v, [ir, jnp.full((L,), h, jnp.int32)])
                v2 = plsc.load_gather(s_v, [ic, jnp.full((L,), H + h, jnp.int32)])
                p = v1 + v2
                e = jnp.exp(jnp.maximum(p, ALPHA * p) - m_v[h])
                e = jnp.where(valid, e, 0.0)
                ew_v[h, pl.ds(ch * CH + g * L, L)] = e
                plsc.store_scatter(
                    esrc_v, [g * L + iota16, jnp.full((L,), h, jnp.int32)], e)
            return 0
        lax.fori_loop(0, CH // L, group_body, 0)
        pltpu.sync_copy(esrc_v, esum_sh.at[row_v.at[ch]], add=True)
        return 0

    lax.fori_loop(0, CH2, chunk_body, 0)

    for h in range(H):
        pltpu.sync_copy(ew_v.at[h], ew_hbm.at[h, pl.ds(w * EPT2, EPT2)])
    plsc.subcore_barrier()
    @pl.when(s < 10)
    def _():
        pltpu.sync_copy(esum_sh.at[pl.ds(s * 1000, 1000)], efl_v)
        pltpu.sync_copy(efl_v, esum_hbm.at[c, pl.ds(s * 1000, 1000)])


def _k2(S, row2, col2, m_bcast, zeros8):
    mesh = plsc.VectorSubcoreMesh(core_axis_name="c", subcore_axis_name="s")
    f = functools.partial(
        pl.kernel,
        out_type=[
            jax.ShapeDtypeStruct((H, E_PAD), jnp.float32),
            jax.ShapeDtypeStruct((NC, N, 2 * H), jnp.float32),
        ],
        mesh=mesh,
        compiler_params=pltpu.CompilerParams(needs_layout_passes=False, use_tc_tiling_on_sc=False),
        scratch_types=[
            pltpu.VMEM((N, 2 * H), jnp.float32),
            pltpu.VMEM((CH2, CH), jnp.int32),
            pltpu.VMEM((CH2, CH), jnp.int32),
            pltpu.VMEM((H, L), jnp.float32),
            pltpu.VMEM((H, EPT2), jnp.float32),
            pltpu.VMEM((CH, 2 * H), jnp.float32),
            pltpu.VMEM((1000, 2 * H), jnp.float32),
            pltpu.VMEM_SHARED((N, 2 * H), jnp.float32),
        ],
    )(_k2_body)
    return f(S, row2, col2, m_bcast, zeros8)


def _k3_body(wh_hbm, row2_hbm, col2_hbm, ew_hbm, z128_hbm,
             out_hbm,
             row_v, col_v, e_v, rb0, rb1, sg0, sg1, ss0, ss1, acc_sh):
    c = lax.axis_index("c")
    s = lax.axis_index("s")

    pltpu.sync_copy(row2_hbm.at[pl.ds(s * CH3, CH3)], row_v)
    pltpu.sync_copy(col2_hbm.at[pl.ds(s * CH3, CH3)], col_v)

    def scale(rb, ch):
        @plsc.parallel_loop(0, CG, unroll=4)
        def _(i):
            e16 = plsc.load_gather(
                e_v, [jnp.full((L,), ch * CG + i, jnp.int32)])
            for r in range(FB // L):
                sl = pl.ds(r * L, L)
                rb[i, sl] = rb[i, sl] * e16

    for fbi in range(NFB // NC):
        fb = c * (NFB // NC) + fbi
        head = fb // (NFB // H)
        @pl.when(s < 10)
        def _():
            pltpu.sync_copy(z128_hbm.at[pl.ds(s * 1000, 1000)],
                            acc_sh.at[pl.ds(s * 1000, 1000)])
        plsc.subcore_barrier()
        pltpu.sync_copy(ew_hbm.at[head, pl.ds(s * EPT3, EPT3)], e_v)

        pltpu.async_copy(wh_hbm.at[fb].at[col_v.at[0]], rb0, sg0)

        def pair_body(j, _):
            ch0 = j * 2
            ch1 = ch0 + 1
            pltpu.async_copy(wh_hbm.at[fb].at[col_v.at[ch1]], rb1, sg1)
            pltpu.make_async_copy(z128_hbm.at[pl.ds(0, CG)], rb0, sg0).wait()
            scale(rb0, ch0)
            pltpu.sync_copy(rb0, acc_sh.at[row_v.at[ch0]], add=True)
            @pl.when(ch0 + 2 < CH3)
            def _():
                pltpu.async_copy(wh_hbm.at[fb].at[col_v.at[ch0 + 2]], rb0, sg0)
            pltpu.make_async_copy(z128_hbm.at[pl.ds(0, CG)], rb1, sg1).wait()
            scale(rb1, ch1)
            pltpu.sync_copy(rb1, acc_sh.at[row_v.at[ch1]], add=True)
            return 0

        lax.fori_loop(0, CH3 // 2, pair_body, 0)
        plsc.subcore_barrier()
        @pl.when(s < 10)
        def _():
            for j, sz in enumerate([CG] * 15 + [1000 - 15 * CG]):
                r0 = s * 1000 + j * CG
                pltpu.sync_copy(acc_sh.at[pl.ds(r0, sz)], rb0.at[pl.ds(0, sz)])
                pltpu.sync_copy(rb0.at[pl.ds(0, sz)],
                                out_hbm.at[fb, pl.ds(r0, sz)])
        plsc.subcore_barrier()


def _k3(wh, row3, col3, ew, zeros128):
    mesh = plsc.VectorSubcoreMesh(core_axis_name="c", subcore_axis_name="s")
    f = functools.partial(
        pl.kernel,
        out_type=jax.ShapeDtypeStruct((NFB, N, FB), jnp.float32),
        mesh=mesh,
        compiler_params=pltpu.CompilerParams(needs_layout_passes=False, use_tc_tiling_on_sc=False),
        scratch_types=[
            pltpu.VMEM((CH3, CG), jnp.int32),
            pltpu.VMEM((CH3, CG), jnp.int32),
            pltpu.VMEM((EPT3,), jnp.float32),
            pltpu.VMEM((CG, FB), jnp.float32),
            pltpu.VMEM((CG, FB), jnp.float32),
            pltpu.SemaphoreType.DMA,
            pltpu.SemaphoreType.DMA,
            pltpu.SemaphoreType.DMA,
            pltpu.SemaphoreType.DMA,
            pltpu.VMEM_SHARED((N, FB), jnp.float32),
        ],
    )(_k3_body)
    return f(wh, row3, col3, ew, zeros128)


def _k4_body(es_ref, g_ref, b_ref, fw_ref, fbias_ref, hacc_ref, wh_ref, out_ref):
    es = es_ref[0] + es_ref[1]
    cols = []
    for fb in range(NFB):
        head = fb // (NFB // H)
        x = hacc_ref[fb] / (es[:, head:head + 1] + 1e-15) + wh_ref[fb]
        cols.append(x)
    X = jnp.concatenate(cols, axis=1)
    mean = jnp.mean(X, axis=1, keepdims=True)
    xc = X - mean
    var = jnp.mean(xc * xc, axis=1, keepdims=True)
    Xn = xc * lax.rsqrt(var + 1e-5) * g_ref[...] + b_ref[...]
    Y = lax.dot_general(Xn, fw_ref[...], (((1,), (1,)), ((), ())),
                        preferred_element_type=jnp.float32) + fbias_ref[...]
    out_ref[...] = jnp.maximum(Y, 0.0)


def _k4(esum2, gamma, beta, ffn_W, ffn_b, hacc, wh):
    return pl.pallas_call(
        _k4_body,
        grid=(N // R4,),
        in_specs=[
            pl.BlockSpec((2, R4, 2 * H), lambda i: (0, i, 0)),
            pl.BlockSpec((1, HID), lambda i: (0, 0)),
            pl.BlockSpec((1, HID), lambda i: (0, 0)),
            pl.BlockSpec((HID, HID), lambda i: (0, 0)),
            pl.BlockSpec((1, HID), lambda i: (0, 0)),
            pl.BlockSpec((NFB, R4, FB), lambda i: (0, i, 0)),
            pl.BlockSpec((NFB, R4, FB), lambda i: (0, i, 0)),
        ],
        out_specs=pl.BlockSpec((R4, HID), lambda i: (i, 0)),
        out_shape=jax.ShapeDtypeStruct((N, HID), jnp.float32),
    )(esum2, gamma.reshape(1, HID), beta.reshape(1, HID), ffn_W,
      ffn_b.reshape(1, HID), hacc, wh)


def kernel(h, edge_index, W, attn_fc, ln_gamma, ln_beta, ffn_W, ffn_b):
    row = edge_index[0].astype(jnp.int32)
    col = edge_index[1].astype(jnp.int32)
    row2 = jnp.pad(row, (0, E_PAD - E)).reshape(NCH, CH)
    col2 = jnp.pad(col, (0, E_PAD - E)).reshape(NCH, CH)
    a1 = attn_fc[:, :OUT_C, 0]
    a2 = attn_fc[:, OUT_C:, 0]
    A = jnp.zeros((HID, 2 * H), jnp.float32)
    hh = jnp.arange(HID) // OUT_C
    A = A.at[jnp.arange(HID), hh].set(a1.reshape(-1))
    A = A.at[jnp.arange(HID), H + hh].set(a2.reshape(-1))

    wh, S, m8 = _k1(h, W, A)
    m_bcast = jnp.broadcast_to(m8[0, :H, None], (H, L))
    zeros8 = jnp.zeros((N, 2 * H), jnp.float32)
    zeros128 = jnp.zeros((N, FB), jnp.float32)

    ew, esum2 = _k2(S, row2, col2, m_bcast, zeros8)
    row3 = row2.reshape(NCG, CG)
    col3 = col2.reshape(NCG, CG)
    hacc = _k3(wh, row3, col3, ew, zeros128)
    return _k4(esum2, ln_gamma, ln_beta, ffn_W, ffn_b, hacc, wh)


# speedup vs baseline: 1.1669x; 1.0274x over previous
import functools

import jax
import jax.numpy as jnp
from jax import lax
from jax.experimental import pallas as pl
from jax.experimental.pallas import tpu as pltpu
from jax.experimental.pallas import tpu_sc as plsc

N = 10000
E = 160000
IN_C = 256
OUT_C = 256
H = 4
HID = 1024
FB = 128
NFB = HID // FB
ALPHA = 0.2

R1 = 1000
R4 = 1000

NC, NS, L = 2, 16, 16
NW = NC * NS
E_PAD = 163840
CH = 128
NCH = E_PAD // CH
CH2 = NCH // NW
EPT2 = E_PAD // NW
CG = 32
NCG = E_PAD // CG
CH3 = NCG // NS
EPT3 = E_PAD // NS
NB = 4
NPT = N // NS


def _k1_body(h_ref, w_ref, a_ref, wh_ref, s_ref, m_ref, m_acc):
    i = pl.program_id(0)
    x = h_ref[...]
    y = lax.dot_general(x, w_ref[...], (((1,), (1,)), ((), ())),
                        preferred_element_type=jnp.float32)
    for fb in range(NFB):
        wh_ref[fb] = y[:, fb * FB:(fb + 1) * FB]
    s = lax.dot_general(y, a_ref[...], (((1,), (0,)), ((), ())),
                        preferred_element_type=jnp.float32)
    s_ref[...] = s
    bm = jnp.max(s, axis=0, keepdims=True)
    prev = m_acc[...]
    cur = jnp.where(i == 0, bm, jnp.maximum(prev, bm))
    m_acc[...] = cur
    m12 = cur[:, :H] + cur[:, H:]
    mlr = jnp.maximum(m12, ALPHA * m12)
    m_ref[...] = jnp.concatenate([mlr, mlr], axis=1)


def _k1(h, W, A):
    return pl.pallas_call(
        _k1_body,
        grid=(N // R1,),
        in_specs=[
            pl.BlockSpec((R1, IN_C), lambda i: (i, 0)),
            pl.BlockSpec((HID, IN_C), lambda i: (0, 0)),
            pl.BlockSpec((HID, 2 * H), lambda i: (0, 0)),
        ],
        out_specs=[
            pl.BlockSpec((NFB, R1, FB), lambda i: (0, i, 0)),
            pl.BlockSpec((R1, 2 * H), lambda i: (i, 0)),
            pl.BlockSpec((1, 2 * H), lambda i: (0, 0)),
        ],
        out_shape=[
            jax.ShapeDtypeStruct((NFB, N, FB), jnp.float32),
            jax.ShapeDtypeStruct((N, 2 * H), jnp.float32),
            jax.ShapeDtypeStruct((1, 2 * H), jnp.float32),
        ],
        scratch_shapes=[pltpu.VMEM((1, 2 * H), jnp.float32)],
    )(h, W, A)


def _k2_body(s_hbm, row2_hbm, col2_hbm, m_hbm, z8_hbm,
             ew_hbm, esum_hbm,
             s_v, row_v, col_v, m_v, ew_v, esrc_v, efl_v, esum_sh):
    c = lax.axis_index("c")
    s = lax.axis_index("s")
    w = c * NS + s

    pltpu.sync_copy(s_hbm, s_v)
    pltpu.sync_copy(row2_hbm.at[pl.ds(w * CH2, CH2)], row_v)
    pltpu.sync_copy(col2_hbm.at[pl.ds(w * CH2, CH2)], col_v)
    pltpu.sync_copy(m_hbm, m_v)
    pltpu.sync_copy(z8_hbm.at[pl.ds(0, CH)], esrc_v)
    @pl.when(s < 10)
    def _():
        pltpu.sync_copy(z8_hbm.at[pl.ds(s * 1000, 1000)],
                        esum_sh.at[pl.ds(s * 1000, 1000)])
    plsc.subcore_barrier()

    iota16 = lax.iota(jnp.int32, L)

    def chunk_body(ch, _):
        def group_body(g, _):
            ir = row_v[ch, pl.ds(g * L, L)]
            ic = col_v[ch, pl.ds(g * L, L)]
            gid = (w * EPT2 + ch * CH + g * L) + iota16
            valid = gid < E
            for h in range(H):
                v1 = plsc.load_gather(s_v, [ir, jnp.full((L,), h, jnp.int32)])
                v2 = plsc.load_gather(s_v, [ic, jnp.full((L,), H + h, jnp.int32)])
                p = v1 + v2
                e = jnp.exp(jnp.maximum(p, ALPHA * p) - m_v[h])
                e = jnp.where(valid, e, 0.0)
                ew_v[h, pl.ds(ch * CH + g * L, L)] = e
                plsc.store_scatter(
                    esrc_v, [g * L + iota16, jnp.full((L,), h, jnp.int32)], e)
            return 0
        lax.fori_loop(0, CH // L, group_body, 0)
        pltpu.sync_copy(esrc_v, esum_sh.at[row_v.at[ch]], add=True)
        return 0

    lax.fori_loop(0, CH2, chunk_body, 0)

    for h in range(H):
        pltpu.sync_copy(ew_v.at[h], ew_hbm.at[h, pl.ds(w * EPT2, EPT2)])
    plsc.subcore_barrier()
    @pl.when(s < 10)
    def _():
        pltpu.sync_copy(esum_sh.at[pl.ds(s * 1000, 1000)], efl_v)
        pltpu.sync_copy(efl_v, esum_hbm.at[c, pl.ds(s * 1000, 1000)])


def _k2(S, row2, col2, m_bcast, zeros8):
    mesh = plsc.VectorSubcoreMesh(core_axis_name="c", subcore_axis_name="s")
    f = functools.partial(
        pl.kernel,
        out_type=[
            jax.ShapeDtypeStruct((H, E_PAD), jnp.float32),
            jax.ShapeDtypeStruct((NC, N, 2 * H), jnp.float32),
        ],
        mesh=mesh,
        compiler_params=pltpu.CompilerParams(needs_layout_passes=False, use_tc_tiling_on_sc=False),
        scratch_types=[
            pltpu.VMEM((N, 2 * H), jnp.float32),
            pltpu.VMEM((CH2, CH), jnp.int32),
            pltpu.VMEM((CH2, CH), jnp.int32),
            pltpu.VMEM((H, L), jnp.float32),
            pltpu.VMEM((H, EPT2), jnp.float32),
            pltpu.VMEM((CH, 2 * H), jnp.float32),
            pltpu.VMEM((1000, 2 * H), jnp.float32),
            pltpu.VMEM_SHARED((N, 2 * H), jnp.float32),
        ],
    )(_k2_body)
    return f(S, row2, col2, m_bcast, zeros8)


def _k3_body(wh_hbm, row2_hbm, col2_hbm, ew_hbm, z128_hbm,
             out_hbm,
             row_v, col_v, e_v, rb0, rb1, rb2, rb3,
             sg0, sg1, sg2, sg3, ss0, ss1, ss2, ss3, acc_sh):
    c = lax.axis_index("c")
    s = lax.axis_index("s")
    rbs = [rb0, rb1, rb2, rb3]
    sgs = [sg0, sg1, sg2, sg3]
    sss = [ss0, ss1, ss2, ss3]

    pltpu.sync_copy(row2_hbm.at[pl.ds(s * CH3, CH3)], row_v)
    pltpu.sync_copy(col2_hbm.at[pl.ds(s * CH3, CH3)], col_v)

    def scale(rb, ch):
        @plsc.parallel_loop(0, CG, unroll=4)
        def _(i):
            e16 = plsc.load_gather(
                e_v, [jnp.full((L,), ch * CG + i, jnp.int32)])
            for r in range(FB // L):
                sl = pl.ds(r * L, L)
                rb[i, sl] = rb[i, sl] * e16

    def drain(sem, rb):
        pltpu.make_async_copy(z128_hbm.at[pl.ds(0, CG)], rb, sem).wait()

    for fbi in range(NFB // NC):
        fb = c * (NFB // NC) + fbi
        head = fb // (NFB // H)
        @pl.when(s < 10)
        def _():
            pltpu.sync_copy(z128_hbm.at[pl.ds(s * 1000, 1000)],
                            acc_sh.at[pl.ds(s * 1000, 1000)])
        plsc.subcore_barrier()
        pltpu.sync_copy(ew_hbm.at[head, pl.ds(s * EPT3, EPT3)], e_v)

        for b in range(3):
            pltpu.async_copy(wh_hbm.at[fb].at[col_v.at[b]], rbs[b], sgs[b])

        def quad_body(j, _):
            for b in range(NB):
                ch = j * NB + b
                bp = (b + 3) % NB
                drain(sgs[b], rbs[b])
                scale(rbs[b], ch)
                pltpu.async_copy(rbs[b], acc_sh.at[row_v.at[ch]],
                                 sss[b], add=True)
                if b == 0:
                    @pl.when(j > 0)
                    def _():
                        drain(sss[bp], rbs[bp])
                else:
                    drain(sss[bp], rbs[bp])
                @pl.when(ch + 3 < CH3)
                def _():
                    pltpu.async_copy(wh_hbm.at[fb].at[col_v.at[ch + 3]],
                                     rbs[bp], sgs[bp])
            return 0

        lax.fori_loop(0, CH3 // NB, quad_body, 0)
        drain(sss[(CH3 - 1) % NB], rbs[(CH3 - 1) % NB])
        plsc.subcore_barrier()
        @pl.when(s < 10)
        def _():
            for j, sz in enumerate([CG] * 31 + [1000 - 31 * CG]):
                r0 = s * 1000 + j * CG
                pltpu.sync_copy(acc_sh.at[pl.ds(r0, sz)], rb0.at[pl.ds(0, sz)])
                pltpu.sync_copy(rb0.at[pl.ds(0, sz)],
                                out_hbm.at[fb, pl.ds(r0, sz)])
        plsc.subcore_barrier()


def _k3(wh, row3, col3, ew, zeros128):
    mesh = plsc.VectorSubcoreMesh(core_axis_name="c", subcore_axis_name="s")
    f = functools.partial(
        pl.kernel,
        out_type=jax.ShapeDtypeStruct((NFB, N, FB), jnp.float32),
        mesh=mesh,
        compiler_params=pltpu.CompilerParams(needs_layout_passes=False, use_tc_tiling_on_sc=False),
        scratch_types=[
            pltpu.VMEM((CH3, CG), jnp.int32),
            pltpu.VMEM((CH3, CG), jnp.int32),
            pltpu.VMEM((EPT3,), jnp.float32),
            pltpu.VMEM((CG, FB), jnp.float32),
            pltpu.VMEM((CG, FB), jnp.float32),
            pltpu.VMEM((CG, FB), jnp.float32),
            pltpu.VMEM((CG, FB), jnp.float32),
            pltpu.SemaphoreType.DMA,
            pltpu.SemaphoreType.DMA,
            pltpu.SemaphoreType.DMA,
            pltpu.SemaphoreType.DMA,
            pltpu.SemaphoreType.DMA,
            pltpu.SemaphoreType.DMA,
            pltpu.SemaphoreType.DMA,
            pltpu.SemaphoreType.DMA,
            pltpu.VMEM_SHARED((N, FB), jnp.float32),
        ],
    )(_k3_body)
    return f(wh, row3, col3, ew, zeros128)


def _k4_body(es_ref, g_ref, b_ref, fw_ref, fbias_ref, hacc_ref, wh_ref, out_ref):
    es = es_ref[0] + es_ref[1]
    cols = []
    for fb in range(NFB):
        head = fb // (NFB // H)
        x = hacc_ref[fb] / (es[:, head:head + 1] + 1e-15) + wh_ref[fb]
        cols.append(x)
    X = jnp.concatenate(cols, axis=1)
    mean = jnp.mean(X, axis=1, keepdims=True)
    xc = X - mean
    var = jnp.mean(xc * xc, axis=1, keepdims=True)
    Xn = xc * lax.rsqrt(var + 1e-5) * g_ref[...] + b_ref[...]
    Y = lax.dot_general(Xn, fw_ref[...], (((1,), (1,)), ((), ())),
                        preferred_element_type=jnp.float32) + fbias_ref[...]
    out_ref[...] = jnp.maximum(Y, 0.0)


def _k4(esum2, gamma, beta, ffn_W, ffn_b, hacc, wh):
    return pl.pallas_call(
        _k4_body,
        grid=(N // R4,),
        in_specs=[
            pl.BlockSpec((2, R4, 2 * H), lambda i: (0, i, 0)),
            pl.BlockSpec((1, HID), lambda i: (0, 0)),
            pl.BlockSpec((1, HID), lambda i: (0, 0)),
            pl.BlockSpec((HID, HID), lambda i: (0, 0)),
            pl.BlockSpec((1, HID), lambda i: (0, 0)),
            pl.BlockSpec((NFB, R4, FB), lambda i: (0, i, 0)),
            pl.BlockSpec((NFB, R4, FB), lambda i: (0, i, 0)),
        ],
        out_specs=pl.BlockSpec((R4, HID), lambda i: (i, 0)),
        out_shape=jax.ShapeDtypeStruct((N, HID), jnp.float32),
    )(esum2, gamma.reshape(1, HID), beta.reshape(1, HID), ffn_W,
      ffn_b.reshape(1, HID), hacc, wh)


def kernel(h, edge_index, W, attn_fc, ln_gamma, ln_beta, ffn_W, ffn_b):
    row = edge_index[0].astype(jnp.int32)
    col = edge_index[1].astype(jnp.int32)
    row2 = jnp.pad(row, (0, E_PAD - E)).reshape(NCH, CH)
    col2 = jnp.pad(col, (0, E_PAD - E)).reshape(NCH, CH)
    a1 = attn_fc[:, :OUT_C, 0]
    a2 = attn_fc[:, OUT_C:, 0]
    A = jnp.zeros((HID, 2 * H), jnp.float32)
    hh = jnp.arange(HID) // OUT_C
    A = A.at[jnp.arange(HID), hh].set(a1.reshape(-1))
    A = A.at[jnp.arange(HID), H + hh].set(a2.reshape(-1))

    wh, S, m8 = _k1(h, W, A)
    m_bcast = jnp.broadcast_to(m8[0, :H, None], (H, L))
    zeros8 = jnp.zeros((N, 2 * H), jnp.float32)
    zeros128 = jnp.zeros((N, FB), jnp.float32)

    ew, esum2 = _k2(S, row2, col2, m_bcast, zeros8)
    row3 = row2.reshape(NCG, CG)
    col3 = col2.reshape(NCG, CG)
    hacc = _k3(wh, row3, col3, ew, zeros128)
    return _k4(esum2, ln_gamma, ln_beta, ffn_W, ffn_b, hacc, wh)


# speedup vs baseline: 1.7628x; 1.5106x over previous
import functools

import jax
import jax.numpy as jnp
from jax import lax
from jax.experimental import pallas as pl
from jax.experimental.pallas import tpu as pltpu
from jax.experimental.pallas import tpu_sc as plsc

N = 10000
E = 160000
IN_C = 256
OUT_C = 256
H = 4
HID = 1024
FB = 128
NFB = HID // FB
ALPHA = 0.2

R1 = 1000
R4 = 1000

NC, NS, L = 2, 16, 16
NW = NC * NS
E_PAD = 163840
CH = 128
NCH = E_PAD // CH
CH2 = NCH // NW
EPT2 = E_PAD // NW
CG = 32
NCG = E_PAD // CG
CH3 = NCG // NS
EPT3 = E_PAD // NS
NB = 4
NPT = N // NS


def _k1_body(h_ref, w_ref, a_ref, wh_ref, whb_ref, s_ref, m_ref, m_acc):
    i = pl.program_id(0)
    x = h_ref[...]
    y = lax.dot_general(x, w_ref[...], (((1,), (1,)), ((), ())),
                        preferred_element_type=jnp.float32)
    for fb in range(NFB):
        ysl = y[:, fb * FB:(fb + 1) * FB]
        wh_ref[fb] = ysl
        whb_ref[fb] = ysl.astype(jnp.bfloat16)
    s = lax.dot_general(y, a_ref[...], (((1,), (0,)), ((), ())),
                        preferred_element_type=jnp.float32)
    s_ref[...] = s
    bm = jnp.max(s, axis=0, keepdims=True)
    prev = m_acc[...]
    cur = jnp.where(i == 0, bm, jnp.maximum(prev, bm))
    m_acc[...] = cur
    m12 = cur[:, :H] + cur[:, H:]
    mlr = jnp.maximum(m12, ALPHA * m12)
    m_ref[...] = jnp.concatenate([mlr, mlr], axis=1)


def _k1(h, W, A):
    return pl.pallas_call(
        _k1_body,
        grid=(N // R1,),
        in_specs=[
            pl.BlockSpec((R1, IN_C), lambda i: (i, 0)),
            pl.BlockSpec((HID, IN_C), lambda i: (0, 0)),
            pl.BlockSpec((HID, 2 * H), lambda i: (0, 0)),
        ],
        out_specs=[
            pl.BlockSpec((NFB, R1, FB), lambda i: (0, i, 0)),
            pl.BlockSpec((NFB, R1, FB), lambda i: (0, i, 0)),
            pl.BlockSpec((R1, 2 * H), lambda i: (i, 0)),
            pl.BlockSpec((1, 2 * H), lambda i: (0, 0)),
        ],
        out_shape=[
            jax.ShapeDtypeStruct((NFB, N, FB), jnp.float32),
            jax.ShapeDtypeStruct((NFB, N, FB), jnp.bfloat16),
            jax.ShapeDtypeStruct((N, 2 * H), jnp.float32),
            jax.ShapeDtypeStruct((1, 2 * H), jnp.float32),
        ],
        scratch_shapes=[pltpu.VMEM((1, 2 * H), jnp.float32)],
    )(h, W, A)


def _k2_body(s_hbm, row2_hbm, col2_hbm, m_hbm, z8_hbm,
             ew_hbm, esum_hbm,
             s_v, row_v, col_v, m_v, ew_v, esrc_v, efl_v, esum_sh):
    c = lax.axis_index("c")
    s = lax.axis_index("s")
    w = c * NS + s

    pltpu.sync_copy(s_hbm, s_v)
    pltpu.sync_copy(row2_hbm.at[pl.ds(w * CH2, CH2)], row_v)
    pltpu.sync_copy(col2_hbm.at[pl.ds(w * CH2, CH2)], col_v)
    pltpu.sync_copy(m_hbm, m_v)
    pltpu.sync_copy(z8_hbm.at[pl.ds(0, CH)], esrc_v)
    @pl.when(s < 10)
    def _():
        pltpu.sync_copy(z8_hbm.at[pl.ds(s * 1000, 1000)],
                        esum_sh.at[pl.ds(s * 1000, 1000)])
    plsc.subcore_barrier()

    iota16 = lax.iota(jnp.int32, L)

    def chunk_body(ch, _):
        def group_body(g, _):
            ir = row_v[ch, pl.ds(g * L, L)]
            ic = col_v[ch, pl.ds(g * L, L)]
            gid = (w * EPT2 + ch * CH + g * L) + iota16
            valid = gid < E
            for h in range(H):
                v1 = plsc.load_gather(s_v, [ir, jnp.full((L,), h, jnp.int32)])
                v2 = plsc.load_gather(s_v, [ic, jnp.full((L,), H + h, jnp.int32)])
                p = v1 + v2
                e = jnp.exp(jnp.maximum(p, ALPHA * p) - m_v[h])
                e = jnp.where(valid, e, 0.0)
                ew_v[h, pl.ds(ch * CH + g * L, L)] = e
                plsc.store_scatter(
                    esrc_v, [g * L + iota16, jnp.full((L,), h, jnp.int32)], e)
            return 0
        lax.fori_loop(0, CH // L, group_body, 0)
        pltpu.sync_copy(esrc_v, esum_sh.at[row_v.at[ch]], add=True)
        return 0

    lax.fori_loop(0, CH2, chunk_body, 0)

    for h in range(H):
        pltpu.sync_copy(ew_v.at[h], ew_hbm.at[h, pl.ds(w * EPT2, EPT2)])
    plsc.subcore_barrier()
    @pl.when(s < 10)
    def _():
        pltpu.sync_copy(esum_sh.at[pl.ds(s * 1000, 1000)], efl_v)
        pltpu.sync_copy(efl_v, esum_hbm.at[c, pl.ds(s * 1000, 1000)])


def _k2(S, row2, col2, m_bcast, zeros8):
    mesh = plsc.VectorSubcoreMesh(core_axis_name="c", subcore_axis_name="s")
    f = functools.partial(
        pl.kernel,
        out_type=[
            jax.ShapeDtypeStruct((H, E_PAD), jnp.float32),
            jax.ShapeDtypeStruct((NC, N, 2 * H), jnp.float32),
        ],
        mesh=mesh,
        compiler_params=pltpu.CompilerParams(needs_layout_passes=False, use_tc_tiling_on_sc=False),
        scratch_types=[
            pltpu.VMEM((N, 2 * H), jnp.float32),
            pltpu.VMEM((CH2, CH), jnp.int32),
            pltpu.VMEM((CH2, CH), jnp.int32),
            pltpu.VMEM((H, L), jnp.float32),
            pltpu.VMEM((H, EPT2), jnp.float32),
            pltpu.VMEM((CH, 2 * H), jnp.float32),
            pltpu.VMEM((1000, 2 * H), jnp.float32),
            pltpu.VMEM_SHARED((N, 2 * H), jnp.float32),
        ],
    )(_k2_body)
    return f(S, row2, col2, m_bcast, zeros8)


def _k3_body(wh_hbm, row2_hbm, col2_hbm, ew_hbm, z128_hbm,
             out_hbm,
             row_v, col_v, e_v, rb0, rb1, rb2, rb3, fb0, fb1,
             sg0, sg1, sg2, sg3, ss0, ss1, ss2, ss3, acc_sh):
    c = lax.axis_index("c")
    s = lax.axis_index("s")
    rbs = [rb0, rb1, rb2, rb3]
    fbs = [fb0, fb1]
    sgs = [sg0, sg1, sg2, sg3]
    sss = [ss0, ss1, ss2, ss3]
    iota2 = lax.iota(jnp.int32, L) * 2

    pltpu.sync_copy(row2_hbm.at[pl.ds(s * CH3, CH3)], row_v)
    pltpu.sync_copy(col2_hbm.at[pl.ds(s * CH3, CH3)], col_v)

    def scale(rb, fbuf, ch):
        @plsc.parallel_loop(0, CG, unroll=4)
        def _(i):
            e16 = plsc.load_gather(
                e_v, [jnp.full((L,), ch * CG + i, jnp.int32)])
            ii = jnp.full((L,), i, jnp.int32)
            for r in range(FB // (2 * L)):
                v = rb[i, pl.ds(r * 2 * L, 2 * L)]
                a, b = plsc.unpack(v, format=plsc.PackFormat.INTERLEAVED)
                plsc.store_scatter(fbuf, [ii, r * 2 * L + iota2], a * e16)
                plsc.store_scatter(fbuf, [ii, r * 2 * L + 1 + iota2], b * e16)

    def drain(sem, nbytes_rb):
        pltpu.make_async_copy(z128_hbm.at[pl.ds(0, CG)], nbytes_rb, sem).wait()

    for fbi in range(NFB // NC):
        fb = c * (NFB // NC) + fbi
        head = fb // (NFB // H)
        @pl.when(s < 10)
        def _():
            pltpu.sync_copy(z128_hbm.at[pl.ds(s * 1000, 1000)],
                            acc_sh.at[pl.ds(s * 1000, 1000)])
        plsc.subcore_barrier()
        pltpu.sync_copy(ew_hbm.at[head, pl.ds(s * EPT3, EPT3)], e_v)

        for b in range(3):
            pltpu.async_copy(wh_hbm.at[fb].at[col_v.at[b]], rbs[b], sgs[b])

        def quad_body(j, _):
            for b in range(NB):
                ch = j * NB + b
                bp = (b + 3) % NB
                f = b % 2
                drain(sgs[b], rbs[b])
                if b < 2:
                    @pl.when(j > 0)
                    def _():
                        drain(sss[f], fbs[f])
                else:
                    drain(sss[f], fbs[f])
                scale(rbs[b], fbs[f], ch)
                pltpu.async_copy(fbs[f], acc_sh.at[row_v.at[ch]],
                                 sss[f], add=True)
                @pl.when(ch + 3 < CH3)
                def _():
                    pltpu.async_copy(wh_hbm.at[fb].at[col_v.at[ch + 3]],
                                     rbs[bp], sgs[bp])
            return 0

        lax.fori_loop(0, CH3 // NB, quad_body, 0)
        drain(sss[(CH3 - 1) % 2], fbs[(CH3 - 1) % 2])
        drain(sss[(CH3 - 2) % 2], fbs[(CH3 - 2) % 2])
        plsc.subcore_barrier()
        @pl.when(s < 10)
        def _():
            for j, sz in enumerate([CG] * 31 + [1000 - 31 * CG]):
                r0 = s * 1000 + j * CG
                pltpu.sync_copy(acc_sh.at[pl.ds(r0, sz)], fb0.at[pl.ds(0, sz)])
                pltpu.sync_copy(fb0.at[pl.ds(0, sz)],
                                out_hbm.at[fb, pl.ds(r0, sz)])
        plsc.subcore_barrier()


def _k3(wh, row3, col3, ew, zeros128):
    mesh = plsc.VectorSubcoreMesh(core_axis_name="c", subcore_axis_name="s")
    f = functools.partial(
        pl.kernel,
        out_type=jax.ShapeDtypeStruct((NFB, N, FB), jnp.float32),
        mesh=mesh,
        compiler_params=pltpu.CompilerParams(needs_layout_passes=False, use_tc_tiling_on_sc=False),
        scratch_types=[
            pltpu.VMEM((CH3, CG), jnp.int32),
            pltpu.VMEM((CH3, CG), jnp.int32),
            pltpu.VMEM((EPT3,), jnp.float32),
            pltpu.VMEM((CG, FB), jnp.bfloat16),
            pltpu.VMEM((CG, FB), jnp.bfloat16),
            pltpu.VMEM((CG, FB), jnp.bfloat16),
            pltpu.VMEM((CG, FB), jnp.bfloat16),
            pltpu.VMEM((CG, FB), jnp.float32),
            pltpu.VMEM((CG, FB), jnp.float32),
            pltpu.SemaphoreType.DMA,
            pltpu.SemaphoreType.DMA,
            pltpu.SemaphoreType.DMA,
            pltpu.SemaphoreType.DMA,
            pltpu.SemaphoreType.DMA,
            pltpu.SemaphoreType.DMA,
            pltpu.SemaphoreType.DMA,
            pltpu.SemaphoreType.DMA,
            pltpu.VMEM_SHARED((N, FB), jnp.float32),
        ],
    )(_k3_body)
    return f(wh, row3, col3, ew, zeros128)


def _k4_body(es_ref, g_ref, b_ref, fw_ref, fbias_ref, hacc_ref, wh_ref, out_ref):
    es = es_ref[0] + es_ref[1]
    cols = []
    for fb in range(NFB):
        head = fb // (NFB // H)
        x = hacc_ref[fb] / (es[:, head:head + 1] + 1e-15) + wh_ref[fb]
        cols.append(x)
    X = jnp.concatenate(cols, axis=1)
    mean = jnp.mean(X, axis=1, keepdims=True)
    xc = X - mean
    var = jnp.mean(xc * xc, axis=1, keepdims=True)
    Xn = xc * lax.rsqrt(var + 1e-5) * g_ref[...] + b_ref[...]
    Y = lax.dot_general(Xn, fw_ref[...], (((1,), (1,)), ((), ())),
                        preferred_element_type=jnp.float32) + fbias_ref[...]
    out_ref[...] = jnp.maximum(Y, 0.0)


def _k4(esum2, gamma, beta, ffn_W, ffn_b, hacc, wh):
    return pl.pallas_call(
        _k4_body,
        grid=(N // R4,),
        in_specs=[
            pl.BlockSpec((2, R4, 2 * H), lambda i: (0, i, 0)),
            pl.BlockSpec((1, HID), lambda i: (0, 0)),
            pl.BlockSpec((1, HID), lambda i: (0, 0)),
            pl.BlockSpec((HID, HID), lambda i: (0, 0)),
            pl.BlockSpec((1, HID), lambda i: (0, 0)),
            pl.BlockSpec((NFB, R4, FB), lambda i: (0, i, 0)),
            pl.BlockSpec((NFB, R4, FB), lambda i: (0, i, 0)),
        ],
        out_specs=pl.BlockSpec((R4, HID), lambda i: (i, 0)),
        out_shape=jax.ShapeDtypeStruct((N, HID), jnp.float32),
    )(esum2, gamma.reshape(1, HID), beta.reshape(1, HID), ffn_W,
      ffn_b.reshape(1, HID), hacc, wh)


def kernel(h, edge_index, W, attn_fc, ln_gamma, ln_beta, ffn_W, ffn_b):
    row = edge_index[0].astype(jnp.int32)
    col = edge_index[1].astype(jnp.int32)
    row2 = jnp.pad(row, (0, E_PAD - E)).reshape(NCH, CH)
    col2 = jnp.pad(col, (0, E_PAD - E)).reshape(NCH, CH)
    a1 = attn_fc[:, :OUT_C, 0]
    a2 = attn_fc[:, OUT_C:, 0]
    A = jnp.zeros((HID, 2 * H), jnp.float32)
    hh = jnp.arange(HID) // OUT_C
    A = A.at[jnp.arange(HID), hh].set(a1.reshape(-1))
    A = A.at[jnp.arange(HID), H + hh].set(a2.reshape(-1))

    wh, whb, S, m8 = _k1(h, W, A)
    m_bcast = jnp.broadcast_to(m8[0, :H, None], (H, L))
    zeros8 = jnp.zeros((N, 2 * H), jnp.float32)
    zeros128 = jnp.zeros((N, FB), jnp.float32)

    ew, esum2 = _k2(S, row2, col2, m_bcast, zeros8)
    row3 = row2.reshape(NCG, CG)
    col3 = col2.reshape(NCG, CG)
    hacc = _k3(whb, row3, col3, ew, zeros128)
    return _k4(esum2, ln_gamma, ln_beta, ffn_W, ffn_b, hacc, wh)
